# baseline (speedup 1.0000x reference)
import functools
import jax
import jax.numpy as jnp
import numpy as np
from jax import lax
from jax.experimental import pallas as pl
from jax.experimental.pallas import tpu as pltpu
from jax.experimental.pallas import tpu_sc as plsc

_NP = 10
_IMG = 256
_HALF = _IMG // 2 + 1
_NBINS = _IMG * _HALF
_FC = 256


def _dot(a, b):
    return jax.lax.dot(a, b, precision=jax.lax.Precision.HIGHEST,
                       preferred_element_type=jnp.float32)


def _dft_stats_kernel(x_ref, cm_ref, sm_ref, out_ref):
    x = x_ref[0]
    cm = cm_ref[...]
    sm = sm_ref[...]
    p = _dot(x, cm)
    q = _dot(x, sm)
    fre = _dot(cm, p) - _dot(sm, q)
    fim = -(_dot(cm, q) + _dot(sm, p))
    mag = jnp.sqrt(fre * fre + fim * fim)
    ang = jnp.arctan2(fim, fre)
    col = jax.lax.broadcasted_iota(jnp.int32, (_IMG, _IMG), 1)
    hmask = (col < _HALF).astype(jnp.float32)
    s1 = jnp.sum(mag * hmask)
    s2 = jnp.sum(ang * hmask)
    dc = jnp.sum(x)
    lane = jax.lax.broadcasted_iota(jnp.int32, (1, 128), 1)
    out_ref[0] = jnp.where(
        lane == 0, s1, jnp.where(lane == 1, s2, jnp.where(lane == 2, dc, 0.0)))


def _sigmoid_v(v):
    return 1.0 / (1.0 + jnp.exp(-v))


def _band_mask_v(featv, w1, b1, w2, b2):
    c1 = _sigmoid_v(featv * w1 + b1)
    c2 = _sigmoid_v(featv * w2 + b2)
    ind = jnp.where(c2 > c1, 1.0, 0.0)
    return jnp.where(c1 * ind < 1.0 / _IMG, 1.0, 0.0)


def _epilogue_sc_kernel(stats_hbm, wpack_hbm, whead_hbm, out_hbm,
                        stats_v, wpack_v, whead_v, pooled_v, row_v):
    info = plsc.get_sparse_core_info()
    nc = info.num_cores
    wid = lax.axis_index("s") * nc + lax.axis_index("c")

    pltpu.sync_copy(stats_hbm, stats_v)
    pltpu.sync_copy(wpack_hbm, wpack_v)
    pltpu.sync_copy(whead_hbm, whead_v)

    inv = jnp.float32(1.0 / (_IMG * _IMG))
    zeros16 = jnp.zeros((16,), jnp.int32)

    dnums = lax.GatherDimensionNumbers(
        offset_dims=(), collapsed_slice_dims=(0,), start_index_map=(0,))

    def splat(v, i):
        return lax.gather(v, (zeros16 + i)[:, None], dnums, (1,),
                          mode=lax.GatherScatterMode.PROMISE_IN_BOUNDS)

    for bc in range(12):
        srow = stats_v[pl.ds(bc * 16, 16)]
        feat1 = splat(srow, 0) * (1.0 / _NBINS)
        feat2 = splat(srow, 1) * (1.0 / _NBINS)
        dcv = splat(srow, 2)
        masks = []
        for m, featv in ((0, feat1), (1, feat2)):
            mrow = lambda a, k: wpack_v[pl.ds(((m * 2 + a) * 4 + k) * 16, 16)]
            mx = _band_mask_v(featv, mrow(0, 0), mrow(0, 1),
                              mrow(0, 2), mrow(0, 3))
            my = _band_mask_v(featv, mrow(1, 0), mrow(1, 1),
                              mrow(1, 2), mrow(1, 3))
            masks.append(jnp.minimum(mx + my, 1.0))
        mask1, mask2 = masks
        amp = jnp.abs(dcv) * inv
        negf = jnp.where(dcv < 0.0, 1.0, 0.0)
        cos_d = 1.0 - 2.0 * negf * mask2
        cos_c = 1.0 - 2.0 * negf * (1.0 - mask2)
        pooled_v[pl.ds(bc * 16, 16)] = amp * mask1 * cos_d
        pooled_v[pl.ds((12 + bc) * 16, 16)] = amp * (1.0 - mask1) * cos_c

    def emit_row(r):
        b = r // _NP
        n = r % _NP
        for c in range(3):
            pdrow = pooled_v[pl.ds((b * 3 + c) * 16, 16)]
            pcrow = pooled_v[pl.ds((12 + b * 3 + c) * 16, 16)]
            pd = splat(pdrow, n)
            pc = splat(pcrow, n)
            for k in range(_FC // 16):
                ws = whead_v[pl.ds(c * _FC + k * 16, 16)]
                wg = whead_v[pl.ds((4 + c) * _FC + k * 16, 16)]
                if c == 0:
                    bs = whead_v[pl.ds(3 * _FC + k * 16, 16)]
                    bg = whead_v[pl.ds(7 * _FC + k * 16, 16)]
                    row_v[pl.ds(k * 16, 16)] = bs + pc * ws
                    row_v[pl.ds(_FC + k * 16, 16)] = bg + pd * wg
                else:
                    row_v[pl.ds(k * 16, 16)] = (
                        row_v[pl.ds(k * 16, 16)] + pc * ws)
                    row_v[pl.ds(_FC + k * 16, 16)] = (
                        row_v[pl.ds(_FC + k * 16, 16)] + pd * wg)
        pltpu.sync_copy(row_v, out_hbm.at[r])

    emit_row(wid)

    @pl.when(wid < 8)
    def _():
        emit_row(wid + 32)


def kernel(x, W1, B1, W2, B2, Wsem, bsem, Wgen, bgen):
    B, C, W, H = x.shape
    xi = x.reshape(B * C, W, H)

    idx = jnp.arange(_IMG, dtype=jnp.int32)
    m = (idx[:, None] * idx[None, :]) % _IMG
    theta = (2.0 * np.pi / _IMG) * m.astype(jnp.float32)
    cm = jnp.cos(theta)
    sm = jnp.sin(theta)

    stats = pl.pallas_call(
        _dft_stats_kernel,
        grid=(B * C,),
        in_specs=[
            pl.BlockSpec((1, _IMG, _IMG), lambda i: (i, 0, 0)),
            pl.BlockSpec((_IMG, _IMG), lambda i: (0, 0)),
            pl.BlockSpec((_IMG, _IMG), lambda i: (0, 0)),
        ],
        out_specs=pl.BlockSpec((1, 1, 128), lambda i: (i, 0, 0)),
        out_shape=jax.ShapeDtypeStruct((B * C, 1, 128), jnp.float32),
    )(xi, cm, sm)
    stats_s = stats.reshape(B * C, 128)[:, :16].reshape(-1)

    rows = []
    for Wm, Bm in ((W1, B1), (W2, B2)):
        for a in range(2):
            for arr in (Wm[a, 1], Bm[a, 1], Wm[a, 2], Bm[a, 2]):
                rows.append(jnp.pad(arr, (0, 16 - _NP)))
    wpack = jnp.stack(rows).reshape(-1)
    whead = jnp.concatenate(
        [Wsem, bsem.reshape(1, -1), Wgen, bgen.reshape(1, -1)],
        axis=0).reshape(-1)

    mesh = plsc.VectorSubcoreMesh(core_axis_name="c", subcore_axis_name="s")
    epilogue = pl.kernel(
        _epilogue_sc_kernel,
        mesh=mesh,
        out_type=jax.ShapeDtypeStruct((B * _NP, 2 * _FC), jnp.float32),
        scratch_types=[
            pltpu.VMEM((12 * 16,), jnp.float32),
            pltpu.VMEM((16 * 16,), jnp.float32),
            pltpu.VMEM((8 * _FC,), jnp.float32),
            pltpu.VMEM((24 * 16,), jnp.float32),
            pltpu.VMEM((2 * _FC,), jnp.float32),
        ],
    )
    return epilogue(stats_s, wpack, whead)
